# baseline (speedup 1.0000x reference)
import functools

import jax
import jax.numpy as jnp
from jax import lax
from jax.experimental import pallas as pl
from jax.experimental.pallas import tpu as pltpu
from jax.experimental.pallas import tpu_sc as plsc

N = 10000
E = 320000
D = 128

DH = D // 2
ROWS = 10240
BLK = 128
N_SUB = 16
NBLK = 160
EPT = NBLK * BLK
E_PAD = N_SUB * EPT
TOT_BLK = E_PAD // BLK
NBUF = 2


def _spmm_body(edata_hbm, feat_hbm, out_hbm,
               e0, e1, d0, d1, r0, r1,
               es0, es1, gs0, gs1, ss0, ss1, shmem):
    ebuf = [e0, e1]
    dbuf = [d0, d1]
    rbuf = [r0, r1]
    esem = [es0, es1]
    gsem = [gs0, gs1]
    ssem = [ss0, ss1]

    c = lax.axis_index("c")
    s = lax.axis_index("s")
    blk0 = s * NBLK
    rbase = s * 640

    for z in range(5):
        pltpu.sync_copy(feat_hbm.at[c, pl.ds(rbase + z * 128, 128)], r0)
        pltpu.sync_copy(r0, shmem.at[pl.ds(rbase + z * 128, 128)])

    plsc.subcore_barrier()

    def start_edata(q, i):
        @pl.when(q < NBLK)
        def _():
            pltpu.async_copy(edata_hbm.at[blk0 + q], ebuf[i], esem[i])

    def wait_edata(i):
        pltpu.make_async_copy(edata_hbm.at[0], ebuf[i], esem[i]).wait()

    def start_gather(i):
        pltpu.async_copy(shmem.at[ebuf[i].at[1]], rbuf[i], gsem[i])

    def wait_gather(i):
        pltpu.make_async_copy(shmem.at[ebuf[i].at[1]], rbuf[i],
                              gsem[i]).wait()

    def start_scatter(i):
        pltpu.async_copy(rbuf[i], shmem.at[dbuf[i]], ssem[i], add=True)

    def wait_scatter(i):
        pltpu.make_async_copy(rbuf[i], shmem.at[dbuf[i]], ssem[i]).wait()

    def copy_dst(i):
        for k in range(BLK // 16):
            dbuf[i][pl.ds(k * 16, 16)] = ebuf[i][0, pl.ds(k * 16, 16)]

    def scale(i):
        eb, rb = ebuf[i], rbuf[i]
        zero16 = jnp.zeros((16,), jnp.float32)

        def _grp(g, _):
            w16 = lax.bitcast_convert_type(
                eb[2, pl.ds(g * 16, 16)], jnp.float32)
            for k in range(16):
                e = g * 16 + k
                w = jnp.broadcast_to(w16[k], (16,))
                for j in range(DH // 16):
                    rb[e, pl.ds(DH + j * 16, 16)] = (
                        rb[e, pl.ds(j * 16, 16)] * w)
                    rb[e, pl.ds(j * 16, 16)] = zero16
            return 0
        lax.fori_loop(0, BLK // 16, _grp, 0)

    def prep(q, i):
        @pl.when(q < NBLK)
        def _():
            wait_edata(i)

            @pl.when(q >= NBUF)
            def _():
                wait_scatter(i)
            start_gather(i)
            copy_dst(i)

    def finish(q, i):
        wait_gather(i)
        start_edata(q + NBUF, i)
        start_scatter(i)

    for i in range(NBUF):
        start_edata(i, i)
    prep(0, 0)

    def _iter(t, _):
        base = t * NBUF
        for i in range(NBUF):
            q = base + i
            prep(q + 1, (i + 1) % NBUF)
            finish(q, i)
        return 0
    lax.fori_loop(0, NBLK // NBUF, _iter, 0)

    for i in range(NBUF):
        wait_scatter(i)

    plsc.subcore_barrier()

    for z in range(5):
        pltpu.sync_copy(shmem.at[pl.ds(rbase + z * 128, 128)], r0)
        pltpu.sync_copy(r0, out_hbm.at[c, pl.ds(rbase + z * 128, 128)])


@jax.jit
def _spmm(edata, feat2):
    mesh = plsc.VectorSubcoreMesh(core_axis_name="c", subcore_axis_name="s")
    run = functools.partial(
        pl.kernel,
        mesh=mesh,
        out_type=jax.ShapeDtypeStruct((2, ROWS, D), jnp.float32),
        scratch_types=(
            [pltpu.VMEM((3, BLK), jnp.int32) for _ in range(NBUF)]
            + [pltpu.VMEM((BLK,), jnp.int32) for _ in range(NBUF)]
            + [pltpu.VMEM((BLK, D), jnp.float32) for _ in range(NBUF)]
            + [pltpu.SemaphoreType.DMA for _ in range(3 * NBUF)]
            + [pltpu.VMEM_SHARED((ROWS, D), jnp.float32)]
        ),
    )(_spmm_body)
    out = run(edata, feat2)
    return jnp.concatenate([out[0, :N, DH:], out[1, :N, DH:]], axis=1)


def kernel(edge_index, adj_values, feat):
    dst = edge_index[0].astype(jnp.int32)
    col = edge_index[1].astype(jnp.int32)
    pad = E_PAD - E
    dst = jnp.pad(dst, (0, pad)).reshape(TOT_BLK, BLK)
    col = jnp.pad(col, (0, pad)).reshape(TOT_BLK, BLK)
    wi = lax.bitcast_convert_type(
        jnp.pad(adj_values, (0, pad)), jnp.int32).reshape(TOT_BLK, BLK)
    edata = jnp.stack([dst, col, wi], axis=1)
    fpad = jnp.pad(feat, ((0, ROWS - N), (0, 0)))
    halves = fpad.reshape(ROWS, 2, DH).transpose(1, 0, 2)
    feat2 = jnp.concatenate(
        [halves, jnp.zeros((2, ROWS, DH), jnp.float32)], axis=2)
    return _spmm(edata, feat2)


# speedup vs baseline: 1.1276x; 1.1276x over previous
import functools

import jax
import jax.numpy as jnp
from jax import lax
from jax.experimental import pallas as pl
from jax.experimental.pallas import tpu as pltpu
from jax.experimental.pallas import tpu_sc as plsc

N = 10000
E = 320000
D = 128

DH = D // 2
ROWS = 10240
BLK = 128
N_SUB = 16
NBLK = 160
EPT = NBLK * BLK
E_PAD = N_SUB * EPT
TOT_BLK = E_PAD // BLK
NBUF = 2


def _spmm_body(edata_hbm, feat_hbm, out_hbm,
               e0, e1, d0, d1, r0, r1,
               es0, es1, gs0, gs1, ss0, ss1, shmem):
    ebuf = [e0, e1]
    dbuf = [d0, d1]
    rbuf = [r0, r1]
    esem = [es0, es1]
    gsem = [gs0, gs1]
    ssem = [ss0, ss1]

    c = lax.axis_index("c")
    s = lax.axis_index("s")
    blk0 = s * NBLK
    rbase = s * 640

    for z in range(5):
        pltpu.sync_copy(feat_hbm.at[c, pl.ds(rbase + z * 128, 128)], r0)
        pltpu.sync_copy(r0, shmem.at[pl.ds(rbase + z * 128, 128)])

    plsc.subcore_barrier()

    def start_edata(q, i):
        @pl.when(q < NBLK)
        def _():
            pltpu.async_copy(edata_hbm.at[blk0 + q], ebuf[i], esem[i])

    def wait_edata(i):
        pltpu.make_async_copy(edata_hbm.at[0], ebuf[i], esem[i]).wait()

    def start_gather(i):
        pltpu.async_copy(shmem.at[ebuf[i].at[1]], rbuf[i], gsem[i])

    def wait_gather(i):
        pltpu.make_async_copy(shmem.at[ebuf[i].at[1]], rbuf[i],
                              gsem[i]).wait()

    def start_scatter(i):
        pltpu.async_copy(rbuf[i], shmem.at[dbuf[i]], ssem[i], add=True)

    def wait_scatter(i):
        pltpu.make_async_copy(rbuf[i], shmem.at[dbuf[i]], ssem[i]).wait()

    def copy_dst(i):
        for k in range(BLK // 16):
            dbuf[i][pl.ds(k * 16, 16)] = ebuf[i][0, pl.ds(k * 16, 16)]

    def scale(i):
        eb, rb = ebuf[i], rbuf[i]
        zero16 = jnp.zeros((16,), jnp.float32)

        def _grp(g, _):
            w16 = lax.bitcast_convert_type(
                eb[2, pl.ds(g * 16, 16)], jnp.float32)
            for k in range(16):
                e = g * 16 + k
                w = jnp.broadcast_to(w16[k], (16,))
                for j in range(DH // 16):
                    rb[e, pl.ds(DH + j * 16, 16)] = (
                        rb[e, pl.ds(j * 16, 16)] * w)
                    rb[e, pl.ds(j * 16, 16)] = zero16
            return 0
        lax.fori_loop(0, BLK // 16, _grp, 0)

    def prep(q, i):
        @pl.when(q < NBLK)
        def _():
            wait_edata(i)

            @pl.when(q >= NBUF)
            def _():
                wait_scatter(i)
            start_gather(i)
            copy_dst(i)

    def finish(q, i):
        wait_gather(i)
        scale(i)
        start_edata(q + NBUF, i)
        start_scatter(i)

    for i in range(NBUF):
        start_edata(i, i)
    prep(0, 0)

    def _iter(t, _):
        base = t * NBUF
        for i in range(NBUF):
            q = base + i
            prep(q + 1, (i + 1) % NBUF)
            finish(q, i)
        return 0
    lax.fori_loop(0, NBLK // NBUF, _iter, 0)

    for i in range(NBUF):
        wait_scatter(i)

    plsc.subcore_barrier()

    for z in range(5):
        pltpu.sync_copy(shmem.at[pl.ds(rbase + z * 128, 128)], r0)
        pltpu.sync_copy(r0, out_hbm.at[c, pl.ds(rbase + z * 128, 128)])


@jax.jit
def _spmm(edata, feat2):
    mesh = plsc.VectorSubcoreMesh(core_axis_name="c", subcore_axis_name="s")
    run = functools.partial(
        pl.kernel,
        mesh=mesh,
        out_type=jax.ShapeDtypeStruct((2, ROWS, D), jnp.float32),
        scratch_types=(
            [pltpu.VMEM((3, BLK), jnp.int32) for _ in range(NBUF)]
            + [pltpu.VMEM((BLK,), jnp.int32) for _ in range(NBUF)]
            + [pltpu.VMEM((BLK, D), jnp.float32) for _ in range(NBUF)]
            + [pltpu.SemaphoreType.DMA for _ in range(3 * NBUF)]
            + [pltpu.VMEM_SHARED((ROWS, D), jnp.float32)]
        ),
    )(_spmm_body)
    out = run(edata, feat2)
    return jnp.concatenate([out[0, :N, DH:], out[1, :N, DH:]], axis=1)


def kernel(edge_index, adj_values, feat):
    dst = edge_index[0].astype(jnp.int32)
    col = edge_index[1].astype(jnp.int32)
    pad = E_PAD - E
    dst = jnp.pad(dst, (0, pad)).reshape(TOT_BLK, BLK)
    col = jnp.pad(col, (0, pad)).reshape(TOT_BLK, BLK)
    wi = lax.bitcast_convert_type(
        jnp.pad(adj_values, (0, pad)), jnp.int32).reshape(TOT_BLK, BLK)
    edata = jnp.stack([dst, col, wi], axis=1)
    fpad = jnp.pad(feat, ((0, ROWS - N), (0, 0)))
    halves = fpad.reshape(ROWS, 2, DH).transpose(1, 0, 2)
    feat2 = jnp.concatenate(
        [halves, jnp.zeros((2, ROWS, DH), jnp.float32)], axis=2)
    return _spmm(edata, feat2)
